# baseline (speedup 1.0000x reference)
import jax
import jax.numpy as jnp
from jax import lax
from jax.experimental import pallas as pl

_BLK = 1024


def kernel(table, idx, targets):
    del targets
    V, C = table.shape
    idx_flat = idx.reshape(-1).astype(jnp.int32)
    N = idx_flat.shape[0]
    nb = N // _BLK

    hi = table.astype(jnp.bfloat16)
    idx3 = idx_flat.astype(jnp.int16).reshape(nb, _BLK, 1)

    def body(hi_ref, idx_ref, out_ref):
        ids = idx_ref[0]
        iota = lax.broadcasted_iota(jnp.int16, (_BLK, V), 1)
        oh = jnp.where(iota == ids, jnp.bfloat16(1), jnp.bfloat16(0))
        out_ref[...] = jnp.dot(oh, hi_ref[...],
                               preferred_element_type=jnp.float32)

    return pl.pallas_call(
        body,
        grid=(nb,),
        in_specs=[
            pl.BlockSpec((V, C), lambda i: (0, 0)),
            pl.BlockSpec((1, _BLK, 1), lambda i: (i, 0, 0)),
        ],
        out_specs=pl.BlockSpec((_BLK, C), lambda i: (i, 0)),
        out_shape=jax.ShapeDtypeStruct((N, C), table.dtype),
    )(hi, idx3)


# speedup vs baseline: 1.0014x; 1.0014x over previous
import jax
import jax.numpy as jnp
from jax import lax
from jax.experimental import pallas as pl
from jax.experimental.pallas import tpu as pltpu

_BLK = 1024


def kernel(table, idx, targets):
    del targets
    V, C = table.shape
    idx_flat = idx.reshape(-1).astype(jnp.int32)
    N = idx_flat.shape[0]
    nb = N // _BLK

    hi = table.astype(jnp.bfloat16)
    idx3 = idx_flat.astype(jnp.int16).reshape(nb, _BLK, 1)

    def body(hi_ref, idx_ref, out_ref):
        ids = idx_ref[0]
        iota = lax.broadcasted_iota(jnp.int16, (_BLK, V), 1)
        oh = jnp.where(iota == ids, jnp.bfloat16(1), jnp.bfloat16(0))
        out_ref[...] = jnp.dot(oh, hi_ref[...],
                               preferred_element_type=jnp.float32)

    return pl.pallas_call(
        body,
        grid=(nb,),
        in_specs=[
            pl.BlockSpec((V, C), lambda i: (0, 0)),
            pl.BlockSpec((1, _BLK, 1), lambda i: (i, 0, 0)),
        ],
        out_specs=pl.BlockSpec((_BLK, C), lambda i: (i, 0)),
        out_shape=jax.ShapeDtypeStruct((N, C), table.dtype),
        compiler_params=pltpu.CompilerParams(
            dimension_semantics=("parallel",)),
    )(hi, idx3)
